# baseline (speedup 1.0000x reference)
import functools

import jax
import jax.numpy as jnp
import numpy as np
from jax.experimental import pallas as pl
from jax.experimental.pallas import tpu as pltpu

_SCALES = (1, 2, 4, 8, 16, 32, 64)
_BETA = 0.02
_B = 64
_T = 64
_C = 32
_K = 8192
_NTOK = _B * _T
_CHUNK = 256
_KTILE = 4096
_NKT = _K // _KTILE

_IDX_OFF = tuple(int(_B * (2**q - 1)) for q in range(7))


def _np_upsample_coeffs(s):
    T = _T
    g = T // s
    t = np.arange(T, dtype=np.float32)
    coords = (t + np.float32(0.5)) * np.float32(s / T) - np.float32(0.5)
    coords = np.clip(coords, np.float32(0.0), np.float32(s - 1))
    lo = np.floor(coords).astype(np.int32)
    hi = np.minimum(lo + 1, s - 1)
    w = (coords - lo.astype(np.float32)).astype(np.float32)
    j = (t // g).astype(np.int32)
    cm = np.zeros(T, np.float32)
    c0 = np.zeros(T, np.float32)
    cp = np.zeros(T, np.float32)
    for i in range(T):
        for (src, wt) in ((lo[i], np.float32(1.0) - w[i]), (hi[i], w[i])):
            d = src - j[i]
            if d == -1:
                cm[i] += wt
            elif d == 0:
                c0[i] += wt
            elif d == 1:
                cp[i] += wt
            else:
                raise AssertionError("upsample source outside 3-tap window")
    tile = lambda v: np.tile(v, _B).reshape(_NTOK, 1)
    return tile(cm), tile(c0), tile(cp)


_UPS = {s: _np_upsample_coeffs(s) for s in _SCALES if s != _T}

_t_of_row = np.arange(_NTOK) % _T
_MASK_TFIRST = (_t_of_row != 0).astype(np.float32).reshape(_NTOK, 1)
_MASK_TLAST = (_t_of_row != _T - 1).astype(np.float32).reshape(_NTOK, 1)

_COEF_COL = {}
_coef_cols = [_MASK_TFIRST, _MASK_TLAST]
for _s in _SCALES:
    if _s == _T:
        continue
    _COEF_COL[_s] = len(_coef_cols)
    _coef_cols.extend(_UPS[_s])
_COEFS = np.concatenate(_coef_cols, axis=1)


def _vq_chunk(flat, q, cbT_ref, cb2_scr, cnt_scr):
    r = flat.shape[0]
    f2 = jnp.sum(flat * flat, axis=1, keepdims=True)

    if _NKT == 1:
        cbT_t = cbT_ref[q, 0]
        scores = jnp.dot(flat, cbT_t)
        dist = f2 - 2.0 * scores + cb2_scr[0]
        tmin = jnp.min(dist, axis=1, keepdims=True)
        ti = jax.lax.broadcasted_iota(jnp.int32, dist.shape, 1)
        idx = jnp.min(jnp.where(dist == tmin, ti, jnp.int32(_K)),
                      axis=1, keepdims=True)
        onehot = (ti == idx).astype(jnp.float32)
        z_e = jax.lax.dot_general(onehot, cbT_t, (((1,), (1,)), ((), ())),
                                  precision=jax.lax.Precision.HIGHEST)
        ones_r = jnp.ones((1, r), jnp.float32)
        cnt_scr[0] = cnt_scr[0] + jax.lax.dot_general(
            ones_r, onehot, (((1,), (0,)), ((), ())),
            precision=jax.lax.Precision.HIGHEST)
        return idx, z_e

    def pass_a(kt, carry):
        m, gidx = carry
        cbT_t = cbT_ref[q, kt]
        cb2_t = cb2_scr[kt]
        scores = jnp.dot(flat, cbT_t)
        dist = f2 - 2.0 * scores + cb2_t
        tmin = jnp.min(dist, axis=1, keepdims=True)
        ti = jax.lax.broadcasted_iota(jnp.int32, dist.shape, 1)
        targ = jnp.min(jnp.where(dist == tmin, ti, jnp.int32(_K)),
                       axis=1, keepdims=True) + kt * _KTILE
        take = tmin < m
        return jnp.where(take, tmin, m), jnp.where(take, targ, gidx)

    m0 = jnp.full((r, 1), jnp.inf, jnp.float32)
    i0 = jnp.zeros((r, 1), jnp.int32)
    _, idx = jax.lax.fori_loop(0, _NKT, pass_a, (m0, i0))

    def pass_b(kt, z_e):
        cbT_t = cbT_ref[q, kt]
        ti = jax.lax.broadcasted_iota(jnp.int32, (r, _KTILE), 1)
        onehot = (ti == idx - kt * _KTILE).astype(jnp.float32)
        z_e = z_e + jax.lax.dot_general(
            onehot, cbT_t, (((1,), (1,)), ((), ())),
            precision=jax.lax.Precision.HIGHEST)
        ones_r = jnp.ones((1, r), jnp.float32)
        cnt_scr[kt] = cnt_scr[kt] + jax.lax.dot_general(
            ones_r, onehot, (((1,), (0,)), ((), ())),
            precision=jax.lax.Precision.HIGHEST)
        return z_e

    z_e = jax.lax.fori_loop(0, _NKT, pass_b, jnp.zeros((r, _C), jnp.float32))
    return idx, z_e


def _conv3(x, w_m, w_0, w_p, brow, mask_first, mask_last):
    xm = jnp.concatenate([jnp.zeros((1, _C), jnp.float32), x[:-1]], axis=0)
    xm = xm * mask_first
    xp = jnp.concatenate([x[1:], jnp.zeros((1, _C), jnp.float32)], axis=0)
    xp = xp * mask_last
    y = jnp.dot(xm, w_m) + jnp.dot(x, w_0) + jnp.dot(xp, w_p) + brow
    return jnp.maximum(y, 0.0)


def _kernel_body(zf_ref, cbT_ref, w1_ref, b1_ref, w2_ref, b2_ref,
                 coef_ref, zhat_ref, idx_ref, scal_ref, ze_scr, zdown_scr,
                 cb2_scr, cnt_scr):
    zf = zf_ref[...]
    mask_first = coef_ref[:, 0:1]
    mask_last = coef_ref[:, 1:2]
    z_q = zf
    z_hat = jnp.zeros((_NTOK, _C), jnp.float32)

    usages = []
    perps = []
    losses = []

    for q, s in enumerate(_SCALES):
        g = _T // s
        n_tok = _B * s
        cbT_q = cbT_ref[q]
        cb2_scr[...] = jnp.sum(cbT_q * cbT_q, axis=1, keepdims=True)
        cnt_scr[...] = jnp.zeros((_NKT, 1, _KTILE), jnp.float32)

        if s == _T:
            z_down = z_q
        else:
            z_down = jnp.mean(z_q.reshape(n_tok, g, _C), axis=1)

        base_off = _IDX_OFF[q]
        if n_tok <= _CHUNK:
            idx, z_e = _vq_chunk(z_down, q, cbT_ref, cb2_scr, cnt_scr)
            idx_ref[pl.ds(base_off, n_tok), :] = idx
            ze_scr[pl.ds(0, n_tok), :] = z_e
        else:
            n_chunks = n_tok // _CHUNK
            zdown_scr[pl.ds(0, n_tok), :] = z_down

            def chunk_body(i, carry):
                tok0 = i * _CHUNK
                flat = zdown_scr[pl.ds(tok0, _CHUNK), :]
                idx_c, ze_c = _vq_chunk(flat, q, cbT_ref, cb2_scr, cnt_scr)
                idx_ref[pl.ds(base_off + tok0, _CHUNK), :] = idx_c
                ze_scr[pl.ds(tok0, _CHUNK), :] = ze_c
                return carry

            jax.lax.fori_loop(0, n_chunks, chunk_body, jnp.int32(0))
        counts = cnt_scr[...]
        z_e = ze_scr[pl.ds(0, n_tok), :]

        used = jnp.sum((counts > 0).astype(jnp.float32))
        usages.append(used / _K * 100.0)
        p = counts / n_tok
        plogp = jnp.where(p > 0, p * jnp.log(p), 0.0)
        perps.append(jnp.exp(-jnp.sum(plogp)))

        if s == _T:
            z_up = z_e
        else:
            col = _COEF_COL[s]
            cm = coef_ref[:, col:col + 1]
            c0 = coef_ref[:, col + 1:col + 2]
            cp = coef_ref[:, col + 2:col + 3]
            rep = lambda v: jnp.broadcast_to(
                v.reshape(n_tok, 1, _C), (n_tok, g, _C)).reshape(_NTOK, _C)
            ctr = rep(z_e)
            prv = rep(jnp.concatenate([z_e[:1], z_e[:-1]], axis=0))
            nxt = rep(jnp.concatenate([z_e[1:], z_e[-1:]], axis=0))
            z_up = (cm * prv + c0 * ctr) + cp * nxt

        h = _conv3(z_up, w1_ref[q, 0], w1_ref[q, 1], w1_ref[q, 2], b1_ref[q],
                   mask_first, mask_last)
        z_hat_q = _conv3(h, w2_ref[q, 0], w2_ref[q, 1], w2_ref[q, 2],
                         b2_ref[q], mask_first, mask_last)

        z_hat = z_hat + z_hat_q
        z_q = z_q - z_hat_q

        d = z_hat - zf
        mse = jnp.mean(d * d)
        losses.append(_BETA * mse + mse)

    def _avg7(vals):
        acc = vals[0]
        for v in vals[1:]:
            acc = acc + v
        return acc / len(vals)

    scal_ref[0, 0] = _avg7(usages)
    scal_ref[0, 1] = _avg7(losses)
    scal_ref[0, 2] = _avg7(perps)

    zhat_ref[...] = (z_hat - zf) + zf


@jax.jit
def kernel(z_BCT, codebooks, phi_w1, phi_b1, phi_w2, phi_b2):
    zf = z_BCT.astype(jnp.float32).transpose(0, 2, 1).reshape(_NTOK, _C)
    cbT = codebooks.astype(jnp.float32).transpose(0, 2, 1).reshape(
        7, _C, _NKT, _KTILE).transpose(0, 2, 1, 3)
    w1t = phi_w1.astype(jnp.float32).transpose(0, 3, 2, 1)
    w2t = phi_w2.astype(jnp.float32).transpose(0, 3, 2, 1)
    b1r = phi_b1.astype(jnp.float32).reshape(7, 1, _C)
    b2r = phi_b2.astype(jnp.float32).reshape(7, 1, _C)

    zhat, idx_flat, scal = pl.pallas_call(
        _kernel_body,
        out_shape=[
            jax.ShapeDtypeStruct((_NTOK, _C), jnp.float32),
            jax.ShapeDtypeStruct((_B * 127, 1), jnp.int32),
            jax.ShapeDtypeStruct((1, 8), jnp.float32),
        ],
        out_specs=[
            pl.BlockSpec(memory_space=pltpu.VMEM),
            pl.BlockSpec(memory_space=pltpu.VMEM),
            pl.BlockSpec(memory_space=pltpu.SMEM),
        ],
        scratch_shapes=[pltpu.VMEM((_NTOK, _C), jnp.float32),
                        pltpu.VMEM((_NTOK, _C), jnp.float32),
                        pltpu.VMEM((_NKT, 1, _KTILE), jnp.float32),
                        pltpu.VMEM((_NKT, 1, _KTILE), jnp.float32)],
    )(zf, cbT, w1t, b1r, w2t, b2r, jnp.asarray(_COEFS))

    z_hat_out = zhat.reshape(_B, _T, _C).transpose(0, 2, 1)
    idx_flat = idx_flat.reshape(_B * 127)
    indices = tuple(
        jax.lax.dynamic_slice(idx_flat, (_IDX_OFF[q],), (_B * s,)).reshape(_B, s)
        for q, s in enumerate(_SCALES))
    all_usages = scal[0, 0]
    all_losses = scal[0, 1]
    all_perp = scal[0, 2]
    return (z_hat_out, all_usages, all_losses, all_perp) + indices


# speedup vs baseline: 1.1500x; 1.1500x over previous
import functools

import jax
import jax.numpy as jnp
import numpy as np
from jax.experimental import pallas as pl
from jax.experimental.pallas import tpu as pltpu

_SCALES = (1, 2, 4, 8, 16, 32, 64)
_BETA = 0.02
_B = 64
_T = 64
_C = 32
_K = 8192
_NTOK = _B * _T
_CHUNK = 128
_KTILE = 8192
_NKT = _K // _KTILE

_IDX_OFF = tuple(int(_B * (2**q - 1)) for q in range(7))


def _np_upsample_coeffs(s):
    T = _T
    g = T // s
    t = np.arange(T, dtype=np.float32)
    coords = (t + np.float32(0.5)) * np.float32(s / T) - np.float32(0.5)
    coords = np.clip(coords, np.float32(0.0), np.float32(s - 1))
    lo = np.floor(coords).astype(np.int32)
    hi = np.minimum(lo + 1, s - 1)
    w = (coords - lo.astype(np.float32)).astype(np.float32)
    j = (t // g).astype(np.int32)
    cm = np.zeros(T, np.float32)
    c0 = np.zeros(T, np.float32)
    cp = np.zeros(T, np.float32)
    for i in range(T):
        for (src, wt) in ((lo[i], np.float32(1.0) - w[i]), (hi[i], w[i])):
            d = src - j[i]
            if d == -1:
                cm[i] += wt
            elif d == 0:
                c0[i] += wt
            elif d == 1:
                cp[i] += wt
            else:
                raise AssertionError("upsample source outside 3-tap window")
    tile = lambda v: np.tile(v, _B).reshape(_NTOK, 1)
    return tile(cm), tile(c0), tile(cp)


_UPS = {s: _np_upsample_coeffs(s) for s in _SCALES if s != _T}

_t_of_row = np.arange(_NTOK) % _T
_MASK_TFIRST = (_t_of_row != 0).astype(np.float32).reshape(_NTOK, 1)
_MASK_TLAST = (_t_of_row != _T - 1).astype(np.float32).reshape(_NTOK, 1)

_COEF_COL = {}
_coef_cols = [_MASK_TFIRST, _MASK_TLAST]
for _s in _SCALES:
    if _s == _T:
        continue
    _COEF_COL[_s] = len(_coef_cols)
    _coef_cols.extend(_UPS[_s])
_COEFS = np.concatenate(_coef_cols, axis=1)


def _vq_chunk(flat, q, cbT_ref, cb2_scr, cnt_scr):
    r = flat.shape[0]
    f2 = jnp.sum(flat * flat, axis=1, keepdims=True)

    if _NKT == 1:
        cbT_t = cbT_ref[q, 0]
        scores = jnp.dot(flat, cbT_t)
        dist = f2 - 2.0 * scores + cb2_scr[0]
        tmin = jnp.min(dist, axis=1, keepdims=True)
        ti = jax.lax.broadcasted_iota(jnp.int32, dist.shape, 1)
        idx = jnp.min(jnp.where(dist == tmin, ti, jnp.int32(_K)),
                      axis=1, keepdims=True)
        onehot = (ti == idx).astype(jnp.float32)
        z_e = jax.lax.dot_general(onehot, cbT_t, (((1,), (1,)), ((), ())),
                                  precision=jax.lax.Precision.HIGHEST)
        cnt_scr[0] = cnt_scr[0] + jnp.sum(onehot, axis=0, keepdims=True)
        return idx, z_e

    def pass_a(kt, carry):
        m, gidx = carry
        cbT_t = cbT_ref[q, kt]
        cb2_t = cb2_scr[kt]
        scores = jnp.dot(flat, cbT_t)
        dist = f2 - 2.0 * scores + cb2_t
        tmin = jnp.min(dist, axis=1, keepdims=True)
        ti = jax.lax.broadcasted_iota(jnp.int32, dist.shape, 1)
        targ = jnp.min(jnp.where(dist == tmin, ti, jnp.int32(_K)),
                       axis=1, keepdims=True) + kt * _KTILE
        take = tmin < m
        return jnp.where(take, tmin, m), jnp.where(take, targ, gidx)

    m0 = jnp.full((r, 1), jnp.inf, jnp.float32)
    i0 = jnp.zeros((r, 1), jnp.int32)
    _, idx = jax.lax.fori_loop(0, _NKT, pass_a, (m0, i0))

    def pass_b(kt, z_e):
        cbT_t = cbT_ref[q, kt]
        ti = jax.lax.broadcasted_iota(jnp.int32, (r, _KTILE), 1)
        onehot = (ti == idx - kt * _KTILE).astype(jnp.float32)
        z_e = z_e + jax.lax.dot_general(
            onehot, cbT_t, (((1,), (1,)), ((), ())),
            precision=jax.lax.Precision.HIGHEST)
        cnt_scr[kt] = cnt_scr[kt] + jnp.sum(onehot, axis=0, keepdims=True)
        return z_e

    z_e = jax.lax.fori_loop(0, _NKT, pass_b, jnp.zeros((r, _C), jnp.float32))
    return idx, z_e


def _conv3(x, w_m, w_0, w_p, brow, mask_first, mask_last):
    xm = jnp.concatenate([jnp.zeros((1, _C), jnp.float32), x[:-1]], axis=0)
    xm = xm * mask_first
    xp = jnp.concatenate([x[1:], jnp.zeros((1, _C), jnp.float32)], axis=0)
    xp = xp * mask_last
    y = jnp.dot(xm, w_m) + jnp.dot(x, w_0) + jnp.dot(xp, w_p) + brow
    return jnp.maximum(y, 0.0)


def _kernel_body(zf_ref, cbT_ref, w1_ref, b1_ref, w2_ref, b2_ref,
                 coef_ref, zhat_ref, idx_ref, scal_ref, ze_scr, zdown_scr,
                 cb2_scr, cnt_scr):
    zf = zf_ref[...]
    mask_first = coef_ref[:, 0:1]
    mask_last = coef_ref[:, 1:2]
    z_q = zf
    z_hat = jnp.zeros((_NTOK, _C), jnp.float32)

    usages = []
    perps = []
    losses = []

    for q, s in enumerate(_SCALES):
        g = _T // s
        n_tok = _B * s
        cbT_q = cbT_ref[q]
        cb2_scr[...] = jnp.sum(cbT_q * cbT_q, axis=1, keepdims=True)
        cnt_scr[...] = jnp.zeros((_NKT, 1, _KTILE), jnp.float32)

        if s == _T:
            z_down = z_q
        else:
            z_down = jnp.mean(z_q.reshape(n_tok, g, _C), axis=1)

        base_off = _IDX_OFF[q]
        if n_tok <= _CHUNK:
            idx, z_e = _vq_chunk(z_down, q, cbT_ref, cb2_scr, cnt_scr)
            idx_ref[pl.ds(base_off, n_tok), :] = idx
            ze_scr[pl.ds(0, n_tok), :] = z_e
        else:
            n_chunks = n_tok // _CHUNK
            zdown_scr[pl.ds(0, n_tok), :] = z_down

            def chunk_body(i, carry):
                tok0 = i * _CHUNK
                flat = zdown_scr[pl.ds(tok0, _CHUNK), :]
                idx_c, ze_c = _vq_chunk(flat, q, cbT_ref, cb2_scr, cnt_scr)
                idx_ref[pl.ds(base_off + tok0, _CHUNK), :] = idx_c
                ze_scr[pl.ds(tok0, _CHUNK), :] = ze_c
                return carry

            jax.lax.fori_loop(0, n_chunks, chunk_body, jnp.int32(0))
        counts = cnt_scr[...]
        z_e = ze_scr[pl.ds(0, n_tok), :]

        used = jnp.sum((counts > 0).astype(jnp.float32))
        usages.append(used / _K * 100.0)
        p = counts / n_tok
        plogp = jnp.where(p > 0, p * jnp.log(p), 0.0)
        perps.append(jnp.exp(-jnp.sum(plogp)))

        if s == _T:
            z_up = z_e
        else:
            col = _COEF_COL[s]
            cm = coef_ref[:, col:col + 1]
            c0 = coef_ref[:, col + 1:col + 2]
            cp = coef_ref[:, col + 2:col + 3]
            rep = lambda v: jnp.broadcast_to(
                v.reshape(n_tok, 1, _C), (n_tok, g, _C)).reshape(_NTOK, _C)
            ctr = rep(z_e)
            prv = rep(jnp.concatenate([z_e[:1], z_e[:-1]], axis=0))
            nxt = rep(jnp.concatenate([z_e[1:], z_e[-1:]], axis=0))
            z_up = (cm * prv + c0 * ctr) + cp * nxt

        h = _conv3(z_up, w1_ref[q, 0], w1_ref[q, 1], w1_ref[q, 2], b1_ref[q],
                   mask_first, mask_last)
        z_hat_q = _conv3(h, w2_ref[q, 0], w2_ref[q, 1], w2_ref[q, 2],
                         b2_ref[q], mask_first, mask_last)

        z_hat = z_hat + z_hat_q
        z_q = z_q - z_hat_q

        d = z_hat - zf
        mse = jnp.mean(d * d)
        losses.append(_BETA * mse + mse)

    def _avg7(vals):
        acc = vals[0]
        for v in vals[1:]:
            acc = acc + v
        return acc / len(vals)

    scal_ref[0, 0] = _avg7(usages)
    scal_ref[0, 1] = _avg7(losses)
    scal_ref[0, 2] = _avg7(perps)

    zhat_ref[...] = (z_hat - zf) + zf


@jax.jit
def kernel(z_BCT, codebooks, phi_w1, phi_b1, phi_w2, phi_b2):
    zf = z_BCT.astype(jnp.float32).transpose(0, 2, 1).reshape(_NTOK, _C)
    cbT = codebooks.astype(jnp.float32).transpose(0, 2, 1).reshape(
        7, _C, _NKT, _KTILE).transpose(0, 2, 1, 3)
    w1t = phi_w1.astype(jnp.float32).transpose(0, 3, 2, 1)
    w2t = phi_w2.astype(jnp.float32).transpose(0, 3, 2, 1)
    b1r = phi_b1.astype(jnp.float32).reshape(7, 1, _C)
    b2r = phi_b2.astype(jnp.float32).reshape(7, 1, _C)

    zhat, idx_flat, scal = pl.pallas_call(
        _kernel_body,
        out_shape=[
            jax.ShapeDtypeStruct((_NTOK, _C), jnp.float32),
            jax.ShapeDtypeStruct((_B * 127, 1), jnp.int32),
            jax.ShapeDtypeStruct((1, 8), jnp.float32),
        ],
        out_specs=[
            pl.BlockSpec(memory_space=pltpu.VMEM),
            pl.BlockSpec(memory_space=pltpu.VMEM),
            pl.BlockSpec(memory_space=pltpu.SMEM),
        ],
        scratch_shapes=[pltpu.VMEM((_NTOK, _C), jnp.float32),
                        pltpu.VMEM((_NTOK, _C), jnp.float32),
                        pltpu.VMEM((_NKT, 1, _KTILE), jnp.float32),
                        pltpu.VMEM((_NKT, 1, _KTILE), jnp.float32)],
    )(zf, cbT, w1t, b1r, w2t, b2r, jnp.asarray(_COEFS))

    z_hat_out = zhat.reshape(_B, _T, _C).transpose(0, 2, 1)
    idx_flat = idx_flat.reshape(_B * 127)
    indices = tuple(
        jax.lax.dynamic_slice(idx_flat, (_IDX_OFF[q],), (_B * s,)).reshape(_B, s)
        for q, s in enumerate(_SCALES))
    all_usages = scal[0, 0]
    all_losses = scal[0, 1]
    all_perp = scal[0, 2]
    return (z_hat_out, all_usages, all_losses, all_perp) + indices


# speedup vs baseline: 1.2279x; 1.0677x over previous
import functools

import jax
import jax.numpy as jnp
import numpy as np
from jax.experimental import pallas as pl
from jax.experimental.pallas import tpu as pltpu

_SCALES = (1, 2, 4, 8, 16, 32, 64)
_BETA = 0.02
_B = 64
_T = 64
_C = 32
_K = 8192
_NTOK = _B * _T
_CHUNK = 512
_KTILE = 2048
_NKT = _K // _KTILE

_IDX_OFF = tuple(int(_B * (2**q - 1)) for q in range(7))


def _np_upsample_coeffs(s):
    T = _T
    g = T // s
    t = np.arange(T, dtype=np.float32)
    coords = (t + np.float32(0.5)) * np.float32(s / T) - np.float32(0.5)
    coords = np.clip(coords, np.float32(0.0), np.float32(s - 1))
    lo = np.floor(coords).astype(np.int32)
    hi = np.minimum(lo + 1, s - 1)
    w = (coords - lo.astype(np.float32)).astype(np.float32)
    j = (t // g).astype(np.int32)
    cm = np.zeros(T, np.float32)
    c0 = np.zeros(T, np.float32)
    cp = np.zeros(T, np.float32)
    for i in range(T):
        for (src, wt) in ((lo[i], np.float32(1.0) - w[i]), (hi[i], w[i])):
            d = src - j[i]
            if d == -1:
                cm[i] += wt
            elif d == 0:
                c0[i] += wt
            elif d == 1:
                cp[i] += wt
            else:
                raise AssertionError("upsample source outside 3-tap window")
    tile = lambda v: np.tile(v, _B).reshape(_NTOK, 1)
    return tile(cm), tile(c0), tile(cp)


_UPS = {s: _np_upsample_coeffs(s) for s in _SCALES if s != _T}

_t_of_row = np.arange(_NTOK) % _T
_MASK_TFIRST = (_t_of_row != 0).astype(np.float32).reshape(_NTOK, 1)
_MASK_TLAST = (_t_of_row != _T - 1).astype(np.float32).reshape(_NTOK, 1)

_COEF_COL = {}
_coef_cols = [_MASK_TFIRST, _MASK_TLAST]
for _s in _SCALES:
    if _s == _T:
        continue
    _COEF_COL[_s] = len(_coef_cols)
    _coef_cols.extend(_UPS[_s])
_COEFS = np.concatenate(_coef_cols, axis=1)


def _vq_chunk(flat, q, cbT_ref, cb2_scr, cnt_scr):
    r = flat.shape[0]
    f2 = jnp.sum(flat * flat, axis=1, keepdims=True)

    if _NKT == 1:
        cbT_t = cbT_ref[q, 0]
        scores = jnp.dot(flat, cbT_t)
        dist = f2 - 2.0 * scores + cb2_scr[0]
        tmin = jnp.min(dist, axis=1, keepdims=True)
        ti = jax.lax.broadcasted_iota(jnp.int32, dist.shape, 1)
        idx = jnp.min(jnp.where(dist == tmin, ti, jnp.int32(_K)),
                      axis=1, keepdims=True)
        onehot = (ti == idx).astype(jnp.float32)
        z_e = jax.lax.dot_general(onehot, cbT_t, (((1,), (1,)), ((), ())),
                                  precision=jax.lax.Precision.HIGHEST)
        cnt_scr[0] = cnt_scr[0] + jnp.sum(onehot, axis=0, keepdims=True)
        return idx, z_e

    def pass_a(kt, carry):
        m, gidx = carry
        cbT_t = cbT_ref[q, kt]
        cb2_t = cb2_scr[kt]
        scores = jnp.dot(flat, cbT_t)
        dist = f2 - 2.0 * scores + cb2_t
        tmin = jnp.min(dist, axis=1, keepdims=True)
        ti = jax.lax.broadcasted_iota(jnp.int32, dist.shape, 1)
        targ = jnp.min(jnp.where(dist == tmin, ti, jnp.int32(_K)),
                       axis=1, keepdims=True) + kt * _KTILE
        take = tmin < m
        return jnp.where(take, tmin, m), jnp.where(take, targ, gidx)

    m0 = jnp.full((r, 1), jnp.inf, jnp.float32)
    i0 = jnp.zeros((r, 1), jnp.int32)
    _, idx = jax.lax.fori_loop(0, _NKT, pass_a, (m0, i0))

    def pass_b(kt, z_e):
        cbT_t = cbT_ref[q, kt]
        ti = jax.lax.broadcasted_iota(jnp.int32, (r, _KTILE), 1)
        onehot = (ti == idx - kt * _KTILE).astype(jnp.float32)
        z_e = z_e + jax.lax.dot_general(
            onehot, cbT_t, (((1,), (1,)), ((), ())),
            precision=jax.lax.Precision.HIGHEST)
        cnt_scr[kt] = cnt_scr[kt] + jnp.sum(onehot, axis=0, keepdims=True)
        return z_e

    z_e = jax.lax.fori_loop(0, _NKT, pass_b, jnp.zeros((r, _C), jnp.float32))
    return idx, z_e


def _conv3(x, w_m, w_0, w_p, brow, mask_first, mask_last):
    xm = jnp.concatenate([jnp.zeros((1, _C), jnp.float32), x[:-1]], axis=0)
    xm = xm * mask_first
    xp = jnp.concatenate([x[1:], jnp.zeros((1, _C), jnp.float32)], axis=0)
    xp = xp * mask_last
    y = jnp.dot(xm, w_m) + jnp.dot(x, w_0) + jnp.dot(xp, w_p) + brow
    return jnp.maximum(y, 0.0)


def _kernel_body(zf_ref, cbT_ref, w1_ref, b1_ref, w2_ref, b2_ref,
                 coef_ref, zhat_ref, idx_ref, scal_ref, ze_scr, zdown_scr,
                 cb2_scr, cnt_scr):
    zf = zf_ref[...]
    mask_first = coef_ref[:, 0:1]
    mask_last = coef_ref[:, 1:2]
    z_q = zf
    z_hat = jnp.zeros((_NTOK, _C), jnp.float32)

    usages = []
    perps = []
    losses = []

    for q, s in enumerate(_SCALES):
        g = _T // s
        n_tok = _B * s
        cbT_q = cbT_ref[q]
        cb2_scr[...] = jnp.sum(cbT_q * cbT_q, axis=1, keepdims=True)
        cnt_scr[...] = jnp.zeros((_NKT, 1, _KTILE), jnp.float32)

        if s == _T:
            z_down = z_q
        else:
            z_down = jnp.mean(z_q.reshape(n_tok, g, _C), axis=1)

        base_off = _IDX_OFF[q]
        if n_tok <= _CHUNK:
            idx, z_e = _vq_chunk(z_down, q, cbT_ref, cb2_scr, cnt_scr)
            idx_ref[pl.ds(base_off, n_tok), :] = idx
            ze_scr[pl.ds(0, n_tok), :] = z_e
        else:
            n_chunks = n_tok // _CHUNK
            zdown_scr[pl.ds(0, n_tok), :] = z_down

            def chunk_body(i, carry):
                tok0 = i * _CHUNK
                flat = zdown_scr[pl.ds(tok0, _CHUNK), :]
                idx_c, ze_c = _vq_chunk(flat, q, cbT_ref, cb2_scr, cnt_scr)
                idx_ref[pl.ds(base_off + tok0, _CHUNK), :] = idx_c
                ze_scr[pl.ds(tok0, _CHUNK), :] = ze_c
                return carry

            jax.lax.fori_loop(0, n_chunks, chunk_body, jnp.int32(0))
        counts = cnt_scr[...]
        z_e = ze_scr[pl.ds(0, n_tok), :]

        used = jnp.sum((counts > 0).astype(jnp.float32))
        usages.append(used / _K * 100.0)
        p = counts / n_tok
        plogp = jnp.where(p > 0, p * jnp.log(p), 0.0)
        perps.append(jnp.exp(-jnp.sum(plogp)))

        if s == _T:
            z_up = z_e
        else:
            col = _COEF_COL[s]
            cm = coef_ref[:, col:col + 1]
            c0 = coef_ref[:, col + 1:col + 2]
            cp = coef_ref[:, col + 2:col + 3]
            rep = lambda v: jnp.broadcast_to(
                v.reshape(n_tok, 1, _C), (n_tok, g, _C)).reshape(_NTOK, _C)
            ctr = rep(z_e)
            prv = rep(jnp.concatenate([z_e[:1], z_e[:-1]], axis=0))
            nxt = rep(jnp.concatenate([z_e[1:], z_e[-1:]], axis=0))
            z_up = (cm * prv + c0 * ctr) + cp * nxt

        h = _conv3(z_up, w1_ref[q, 0], w1_ref[q, 1], w1_ref[q, 2], b1_ref[q],
                   mask_first, mask_last)
        z_hat_q = _conv3(h, w2_ref[q, 0], w2_ref[q, 1], w2_ref[q, 2],
                         b2_ref[q], mask_first, mask_last)

        z_hat = z_hat + z_hat_q
        z_q = z_q - z_hat_q

        d = z_hat - zf
        mse = jnp.mean(d * d)
        losses.append(_BETA * mse + mse)

    def _avg7(vals):
        acc = vals[0]
        for v in vals[1:]:
            acc = acc + v
        return acc / len(vals)

    scal_ref[0, 0] = _avg7(usages)
    scal_ref[0, 1] = _avg7(losses)
    scal_ref[0, 2] = _avg7(perps)

    zhat_ref[...] = (z_hat - zf) + zf


@jax.jit
def kernel(z_BCT, codebooks, phi_w1, phi_b1, phi_w2, phi_b2):
    zf = z_BCT.astype(jnp.float32).transpose(0, 2, 1).reshape(_NTOK, _C)
    cbT = codebooks.astype(jnp.float32).transpose(0, 2, 1).reshape(
        7, _C, _NKT, _KTILE).transpose(0, 2, 1, 3)
    w1t = phi_w1.astype(jnp.float32).transpose(0, 3, 2, 1)
    w2t = phi_w2.astype(jnp.float32).transpose(0, 3, 2, 1)
    b1r = phi_b1.astype(jnp.float32).reshape(7, 1, _C)
    b2r = phi_b2.astype(jnp.float32).reshape(7, 1, _C)

    zhat, idx_flat, scal = pl.pallas_call(
        _kernel_body,
        out_shape=[
            jax.ShapeDtypeStruct((_NTOK, _C), jnp.float32),
            jax.ShapeDtypeStruct((_B * 127, 1), jnp.int32),
            jax.ShapeDtypeStruct((1, 8), jnp.float32),
        ],
        out_specs=[
            pl.BlockSpec(memory_space=pltpu.VMEM),
            pl.BlockSpec(memory_space=pltpu.VMEM),
            pl.BlockSpec(memory_space=pltpu.SMEM),
        ],
        scratch_shapes=[pltpu.VMEM((_NTOK, _C), jnp.float32),
                        pltpu.VMEM((_NTOK, _C), jnp.float32),
                        pltpu.VMEM((_NKT, 1, _KTILE), jnp.float32),
                        pltpu.VMEM((_NKT, 1, _KTILE), jnp.float32)],
    )(zf, cbT, w1t, b1r, w2t, b2r, jnp.asarray(_COEFS))

    z_hat_out = zhat.reshape(_B, _T, _C).transpose(0, 2, 1)
    idx_flat = idx_flat.reshape(_B * 127)
    indices = tuple(
        jax.lax.dynamic_slice(idx_flat, (_IDX_OFF[q],), (_B * s,)).reshape(_B, s)
        for q, s in enumerate(_SCALES))
    all_usages = scal[0, 0]
    all_losses = scal[0, 1]
    all_perp = scal[0, 2]
    return (z_hat_out, all_usages, all_losses, all_perp) + indices


# speedup vs baseline: 1.2581x; 1.0245x over previous
import functools

import jax
import jax.numpy as jnp
import numpy as np
from jax.experimental import pallas as pl
from jax.experimental.pallas import tpu as pltpu

_SCALES = (1, 2, 4, 8, 16, 32, 64)
_BETA = 0.02
_B = 64
_T = 64
_C = 32
_K = 8192
_NTOK = _B * _T
_CHUNK = 256
_KTILE = 4096
_NKT = _K // _KTILE

_IDX_OFF = tuple(int(_B * (2**q - 1)) for q in range(7))


def _np_upsample_coeffs(s):
    T = _T
    g = T // s
    t = np.arange(T, dtype=np.float32)
    coords = (t + np.float32(0.5)) * np.float32(s / T) - np.float32(0.5)
    coords = np.clip(coords, np.float32(0.0), np.float32(s - 1))
    lo = np.floor(coords).astype(np.int32)
    hi = np.minimum(lo + 1, s - 1)
    w = (coords - lo.astype(np.float32)).astype(np.float32)
    j = (t // g).astype(np.int32)
    cm = np.zeros(T, np.float32)
    c0 = np.zeros(T, np.float32)
    cp = np.zeros(T, np.float32)
    for i in range(T):
        for (src, wt) in ((lo[i], np.float32(1.0) - w[i]), (hi[i], w[i])):
            d = src - j[i]
            if d == -1:
                cm[i] += wt
            elif d == 0:
                c0[i] += wt
            elif d == 1:
                cp[i] += wt
            else:
                raise AssertionError("upsample source outside 3-tap window")
    tile = lambda v: np.tile(v, _B).reshape(_NTOK, 1)
    return tile(cm), tile(c0), tile(cp)


_UPS = {s: _np_upsample_coeffs(s) for s in _SCALES if s != _T}

_t_of_row = np.arange(_NTOK) % _T
_MASK_TFIRST = (_t_of_row != 0).astype(np.float32).reshape(_NTOK, 1)
_MASK_TLAST = (_t_of_row != _T - 1).astype(np.float32).reshape(_NTOK, 1)

_COEF_COL = {}
_coef_cols = [_MASK_TFIRST, _MASK_TLAST]
for _s in _SCALES:
    if _s == _T:
        continue
    _COEF_COL[_s] = len(_coef_cols)
    _coef_cols.extend(_UPS[_s])
_COEFS = np.concatenate(_coef_cols, axis=1)


def _vq_chunk(flat, q, cbT_ref, cb2_scr, cnt_scr):
    r = flat.shape[0]
    f2 = jnp.sum(flat * flat, axis=1, keepdims=True)

    if _NKT == 1:
        cbT_t = cbT_ref[q, 0]
        scores = jnp.dot(flat, cbT_t)
        dist = f2 - 2.0 * scores + cb2_scr[0]
        tmin = jnp.min(dist, axis=1, keepdims=True)
        ti = jax.lax.broadcasted_iota(jnp.int32, dist.shape, 1)
        idx = jnp.min(jnp.where(dist == tmin, ti, jnp.int32(_K)),
                      axis=1, keepdims=True)
        onehot = (ti == idx).astype(jnp.float32)
        z_e = jax.lax.dot_general(onehot, cbT_t, (((1,), (1,)), ((), ())),
                                  precision=jax.lax.Precision.HIGHEST)
        cnt_scr[0] = cnt_scr[0] + jnp.sum(onehot, axis=0, keepdims=True)
        return idx, z_e

    def pass_a(kt, carry):
        m, gidx = carry
        cbT_t = cbT_ref[q, kt]
        cb2_t = cb2_scr[kt]
        scores = jnp.dot(flat, cbT_t)
        dist = f2 - 2.0 * scores + cb2_t
        tmin = jnp.min(dist, axis=1, keepdims=True)
        ti = jax.lax.broadcasted_iota(jnp.int32, dist.shape, 1)
        targ = jnp.min(jnp.where(dist == tmin, ti, jnp.int32(_K)),
                       axis=1, keepdims=True) + kt * _KTILE
        take = tmin < m
        return jnp.where(take, tmin, m), jnp.where(take, targ, gidx)

    m0 = jnp.full((r, 1), jnp.inf, jnp.float32)
    i0 = jnp.zeros((r, 1), jnp.int32)
    _, idx = jax.lax.fori_loop(0, _NKT, pass_a, (m0, i0))

    def pass_b(kt, z_e):
        cbT_t = cbT_ref[q, kt]
        ti = jax.lax.broadcasted_iota(jnp.int32, (r, _KTILE), 1)
        onehot = (ti == idx - kt * _KTILE).astype(jnp.float32)
        z_e = z_e + jax.lax.dot_general(
            onehot, cbT_t, (((1,), (1,)), ((), ())),
            precision=jax.lax.Precision.HIGHEST)
        cnt_scr[kt] = cnt_scr[kt] + jnp.sum(onehot, axis=0, keepdims=True)
        return z_e

    z_e = jax.lax.fori_loop(0, _NKT, pass_b, jnp.zeros((r, _C), jnp.float32))
    return idx, z_e


def _conv3(x, w_m, w_0, w_p, brow, mask_first, mask_last):
    xm = jnp.concatenate([jnp.zeros((1, _C), jnp.float32), x[:-1]], axis=0)
    xm = xm * mask_first
    xp = jnp.concatenate([x[1:], jnp.zeros((1, _C), jnp.float32)], axis=0)
    xp = xp * mask_last
    y = jnp.dot(xm, w_m) + jnp.dot(x, w_0) + jnp.dot(xp, w_p) + brow
    return jnp.maximum(y, 0.0)


def _kernel_body(zf_ref, cbT_ref, w1_ref, b1_ref, w2_ref, b2_ref,
                 coef_ref, zhat_ref, idx_ref, scal_ref, ze_scr, zdown_scr,
                 cb2_scr, cnt_scr):
    zf = zf_ref[...]
    mask_first = coef_ref[:, 0:1]
    mask_last = coef_ref[:, 1:2]
    z_q = zf
    z_hat = jnp.zeros((_NTOK, _C), jnp.float32)

    usages = []
    perps = []
    losses = []

    for q, s in enumerate(_SCALES):
        g = _T // s
        n_tok = _B * s
        cbT_q = cbT_ref[q]
        cb2_scr[...] = jnp.sum(cbT_q * cbT_q, axis=1, keepdims=True)
        cnt_scr[...] = jnp.zeros((_NKT, 1, _KTILE), jnp.float32)

        if s == _T:
            z_down = z_q
        else:
            z_down = jnp.mean(z_q.reshape(n_tok, g, _C), axis=1)

        base_off = _IDX_OFF[q]
        if n_tok <= _CHUNK:
            idx, z_e = _vq_chunk(z_down, q, cbT_ref, cb2_scr, cnt_scr)
            idx_ref[pl.ds(base_off, n_tok), :] = idx
            ze_scr[pl.ds(0, n_tok), :] = z_e
        else:
            n_chunks = n_tok // _CHUNK
            zdown_scr[pl.ds(0, n_tok), :] = z_down

            def chunk_body(i, carry):
                tok0 = i * _CHUNK
                flat = zdown_scr[pl.ds(tok0, _CHUNK), :]
                idx_c, ze_c = _vq_chunk(flat, q, cbT_ref, cb2_scr, cnt_scr)
                idx_ref[pl.ds(base_off + tok0, _CHUNK), :] = idx_c
                ze_scr[pl.ds(tok0, _CHUNK), :] = ze_c
                return carry

            jax.lax.fori_loop(0, n_chunks, chunk_body, jnp.int32(0),
                              unroll=2)
        counts = cnt_scr[...]
        z_e = ze_scr[pl.ds(0, n_tok), :]

        used = jnp.sum((counts > 0).astype(jnp.float32))
        usages.append(used / _K * 100.0)
        p = counts / n_tok
        plogp = jnp.where(p > 0, p * jnp.log(p), 0.0)
        perps.append(jnp.exp(-jnp.sum(plogp)))

        if s == _T:
            z_up = z_e
        else:
            col = _COEF_COL[s]
            cm = coef_ref[:, col:col + 1]
            c0 = coef_ref[:, col + 1:col + 2]
            cp = coef_ref[:, col + 2:col + 3]
            rep = lambda v: jnp.broadcast_to(
                v.reshape(n_tok, 1, _C), (n_tok, g, _C)).reshape(_NTOK, _C)
            ctr = rep(z_e)
            prv = rep(jnp.concatenate([z_e[:1], z_e[:-1]], axis=0))
            nxt = rep(jnp.concatenate([z_e[1:], z_e[-1:]], axis=0))
            z_up = (cm * prv + c0 * ctr) + cp * nxt

        h = _conv3(z_up, w1_ref[q, 0], w1_ref[q, 1], w1_ref[q, 2], b1_ref[q],
                   mask_first, mask_last)
        z_hat_q = _conv3(h, w2_ref[q, 0], w2_ref[q, 1], w2_ref[q, 2],
                         b2_ref[q], mask_first, mask_last)

        z_hat = z_hat + z_hat_q
        z_q = z_q - z_hat_q

        d = z_hat - zf
        mse = jnp.mean(d * d)
        losses.append(_BETA * mse + mse)

    def _avg7(vals):
        acc = vals[0]
        for v in vals[1:]:
            acc = acc + v
        return acc / len(vals)

    scal_ref[0, 0] = _avg7(usages)
    scal_ref[0, 1] = _avg7(losses)
    scal_ref[0, 2] = _avg7(perps)

    zhat_ref[...] = (z_hat - zf) + zf


@jax.jit
def kernel(z_BCT, codebooks, phi_w1, phi_b1, phi_w2, phi_b2):
    zf = z_BCT.astype(jnp.float32).transpose(0, 2, 1).reshape(_NTOK, _C)
    cbT = codebooks.astype(jnp.float32).transpose(0, 2, 1).reshape(
        7, _C, _NKT, _KTILE).transpose(0, 2, 1, 3)
    w1t = phi_w1.astype(jnp.float32).transpose(0, 3, 2, 1)
    w2t = phi_w2.astype(jnp.float32).transpose(0, 3, 2, 1)
    b1r = phi_b1.astype(jnp.float32).reshape(7, 1, _C)
    b2r = phi_b2.astype(jnp.float32).reshape(7, 1, _C)

    zhat, idx_flat, scal = pl.pallas_call(
        _kernel_body,
        out_shape=[
            jax.ShapeDtypeStruct((_NTOK, _C), jnp.float32),
            jax.ShapeDtypeStruct((_B * 127, 1), jnp.int32),
            jax.ShapeDtypeStruct((1, 8), jnp.float32),
        ],
        out_specs=[
            pl.BlockSpec(memory_space=pltpu.VMEM),
            pl.BlockSpec(memory_space=pltpu.VMEM),
            pl.BlockSpec(memory_space=pltpu.SMEM),
        ],
        scratch_shapes=[pltpu.VMEM((_NTOK, _C), jnp.float32),
                        pltpu.VMEM((_NTOK, _C), jnp.float32),
                        pltpu.VMEM((_NKT, 1, _KTILE), jnp.float32),
                        pltpu.VMEM((_NKT, 1, _KTILE), jnp.float32)],
    )(zf, cbT, w1t, b1r, w2t, b2r, jnp.asarray(_COEFS))

    z_hat_out = zhat.reshape(_B, _T, _C).transpose(0, 2, 1)
    idx_flat = idx_flat.reshape(_B * 127)
    indices = tuple(
        jax.lax.dynamic_slice(idx_flat, (_IDX_OFF[q],), (_B * s,)).reshape(_B, s)
        for q, s in enumerate(_SCALES))
    all_usages = scal[0, 0]
    all_losses = scal[0, 1]
    all_perp = scal[0, 2]
    return (z_hat_out, all_usages, all_losses, all_perp) + indices
